# Initial kernel scaffold; baseline (speedup 1.0000x reference)
import jax
import jax.numpy as jnp
from jax.experimental import pallas as pl


def kernel(in_feat, edge_index, W0, b0, W1, b1, Wl0, bl0, Wl2, bl2, Wl3, bl3):
    raise NotImplementedError("write your pallas kernel here")


# speedup vs baseline: 7.1280x; 7.1280x over previous
import jax
import jax.numpy as jnp
from jax.experimental import pallas as pl
from jax.experimental.pallas import tpu as pltpu

N_NODES = 6
N_EDGES = 24
T = 16384
H1, H2, H3 = 128, 256, 128
TT = 2048


def _leaky(x):
    return jnp.where(x >= 0, x, 0.01 * x)


def _bdot(a, b):
    return jnp.dot(a.astype(jnp.bfloat16), b.astype(jnp.bfloat16),
                   preferred_element_type=jnp.float32)


def _fused(edge_ref, x_ref, w0_ref, b0_ref, w1_ref, b1_ref,
           wl0_ref, bl0_ref, wl2_ref, bl2_ref, wl3_ref, bl3_ref, out_ref):
    edges = edge_ref[...]
    src = edges[0:1, :]
    dst = edges[1:2, :]
    iota = jax.lax.broadcasted_iota(jnp.int32, (N_NODES, N_EDGES), 0)
    s_onehot = (src == iota).astype(jnp.float32)
    d_onehot = (dst == iota).astype(jnp.float32)
    count_t = jax.lax.dot_general(
        s_onehot, d_onehot, (((1,), (1,)), ((), ())),
        preferred_element_type=jnp.float32,
        precision=jax.lax.Precision.HIGHEST)
    deg_out = jnp.clip(jnp.sum(s_onehot, axis=1, keepdims=True), 1.0, None)
    deg_in = jnp.clip(jnp.sum(d_onehot, axis=1, keepdims=True), 1.0, None)
    a_t = count_t * jax.lax.rsqrt(deg_out) * jnp.transpose(jax.lax.rsqrt(deg_in))

    x = x_ref[...]
    agg1 = jnp.dot(x, a_t, preferred_element_type=jnp.float32,
                   precision=jax.lax.Precision.HIGHEST)

    w0 = w0_ref[...]
    b0 = b0_ref[...]
    h1 = [_leaky(agg1[:, s:s + 1] * w0 + b0) for s in range(N_NODES)]

    w1 = w1_ref[...]
    cols = []
    for d in range(N_NODES):
        agg2 = h1[0] * a_t[0:1, d:d + 1]
        for s in range(1, N_NODES):
            agg2 = agg2 + h1[s] * a_t[s:s + 1, d:d + 1]
        cols.append(_bdot(agg2, w1))
    h2 = _leaky(jnp.concatenate(cols, axis=1) + b1_ref[...])

    h3 = _leaky(_bdot(h2, wl0_ref[...]) + bl0_ref[...])
    h4 = _leaky(_bdot(h3, wl2_ref[...]) + bl2_ref[...])
    out_ref[...] = _leaky(_bdot(h4, wl3_ref[...]) + bl3_ref[...])


def kernel(in_feat, edge_index, W0, b0, W1, b1, Wl0, bl0, Wl2, bl2, Wl3, bl3):
    x_t = jnp.transpose(in_feat[:, :, 0])
    edge = edge_index.astype(jnp.int32)

    def fixed(*_):
        return (0, 0)

    out = pl.pallas_call(
        _fused,
        grid=(T // TT,),
        in_specs=[
            pl.BlockSpec((2, N_EDGES), fixed),
            pl.BlockSpec((TT, N_NODES), lambda i: (i, 0)),
            pl.BlockSpec((1, H1), fixed),
            pl.BlockSpec((1, H1), fixed),
            pl.BlockSpec((H1, 1), fixed),
            pl.BlockSpec((1, 1), fixed),
            pl.BlockSpec((N_NODES, H2), fixed),
            pl.BlockSpec((1, H2), fixed),
            pl.BlockSpec((H2, H3), fixed),
            pl.BlockSpec((1, H3), fixed),
            pl.BlockSpec((H3, 1), fixed),
            pl.BlockSpec((1, 1), fixed),
        ],
        out_specs=pl.BlockSpec((TT, 1), lambda i: (i, 0)),
        out_shape=jax.ShapeDtypeStruct((T, 1), jnp.float32),
        compiler_params=pltpu.CompilerParams(
            dimension_semantics=("arbitrary",)),
    )(
        edge, x_t,
        W0, b0.reshape(1, H1), W1, b1.reshape(1, 1),
        Wl0, bl0.reshape(1, H2), Wl2, bl2.reshape(1, H3),
        Wl3, bl3.reshape(1, 1),
    )
    return out
